# baseline (speedup 1.0000x reference)
import functools

import jax
import jax.numpy as jnp
from jax import lax
from jax.experimental import pallas as pl
from jax.experimental.pallas import tpu as pltpu
from jax.experimental.pallas import tpu_sc as plsc

EMB = 64
NROW = 4096
NPOS = 50
NC, NS = 2, 16
NW = NC * NS
RPW = NROW // NW
NBUF = 4
NG = RPW // NBUF

_mesh = plsc.VectorSubcoreMesh(core_axis_name="c", subcore_axis_name="s")


@functools.partial(
    pl.kernel,
    mesh=_mesh,
    out_type=jax.ShapeDtypeStruct((NROW, NPOS, EMB), jnp.float32),
    compiler_params=pltpu.CompilerParams(use_tc_tiling_on_sc=False),
    scratch_types=[
        pltpu.VMEM((RPW, NPOS), jnp.int32),
        *[pltpu.VMEM((NPOS, EMB), jnp.float32) for _ in range(NBUF)],
        *[pltpu.SemaphoreType.DMA for _ in range(2 * NBUF)],
    ],
)
def _gather_kernel(idx_hbm, table_hbm, out_hbm, idx_v, *bufs_and_sems):
    rows = bufs_and_sems[:NBUF]
    gsem = bufs_and_sems[NBUF:2 * NBUF]
    wsem = bufs_and_sems[2 * NBUF:]

    wid = lax.axis_index("s") * NC + lax.axis_index("c")
    base = wid * RPW

    pltpu.sync_copy(idx_hbm.at[pl.ds(base, RPW)], idx_v)

    def fire_gather(v, b):
        pltpu.make_async_copy(
            table_hbm.at[idx_v.at[v]], rows[b], gsem[b]).start()

    def wait_gather(v, b):
        pltpu.make_async_copy(
            table_hbm.at[idx_v.at[v]], rows[b], gsem[b]).wait()

    def fire_write(v, b):
        pltpu.make_async_copy(rows[b], out_hbm.at[base + v], wsem[b]).start()

    def wait_write(v, b):
        pltpu.make_async_copy(rows[b], out_hbm.at[base + v], wsem[b]).wait()

    for b in range(NBUF):
        fire_gather(b, b)

    def group_body(g, carry):
        for b in range(NBUF):
            v = g * NBUF + b
            wait_gather(v, b)
            fire_write(v, b)
            wait_write(v, b)
            fire_gather(v + NBUF, b)
        return carry

    lax.fori_loop(0, NG - 1, group_body, 0)

    for b in range(NBUF):
        v = (NG - 1) * NBUF + b
        wait_gather(v, b)
        fire_write(v, b)
    for b in range(NBUF):
        v = (NG - 1) * NBUF + b
        wait_write(v, b)


def kernel(vocab_ids, node_embs):
    return _gather_kernel(vocab_ids.astype(jnp.int32), node_embs)


# speedup vs baseline: 1.0632x; 1.0632x over previous
import functools

import jax
import jax.numpy as jnp
from jax import lax
from jax.experimental import pallas as pl
from jax.experimental.pallas import tpu as pltpu
from jax.experimental.pallas import tpu_sc as plsc

EMB = 64
NROW = 4096
NPOS = 50
NC, NS = 2, 16
NW = NC * NS
RPW = NROW // NW
NBUF = 4
NG = RPW // NBUF

_mesh = plsc.VectorSubcoreMesh(core_axis_name="c", subcore_axis_name="s")


@functools.partial(
    pl.kernel,
    mesh=_mesh,
    out_type=jax.ShapeDtypeStruct((NROW, NPOS, EMB), jnp.float32),
    compiler_params=pltpu.CompilerParams(use_tc_tiling_on_sc=False),
    scratch_types=[
        pltpu.VMEM((RPW, NPOS), jnp.int32),
        *[pltpu.VMEM((NPOS, 2 * EMB), jnp.float32) for _ in range(NBUF)],
        *[pltpu.SemaphoreType.DMA for _ in range(2 * NBUF)],
    ],
)
def _gather_kernel(idx_hbm, table_hbm, out_hbm, idx_v, *bufs_and_sems):
    rows = bufs_and_sems[:NBUF]
    gsem = bufs_and_sems[NBUF:2 * NBUF]
    wsem = bufs_and_sems[2 * NBUF:]

    wid = lax.axis_index("s") * NC + lax.axis_index("c")
    base = wid * RPW

    pltpu.sync_copy(idx_hbm.at[pl.ds(base, RPW)], idx_v)

    def fire_gather(v, b):
        pltpu.make_async_copy(
            table_hbm.at[idx_v.at[v]], rows[b], gsem[b]).start()

    def wait_gather(v, b):
        pltpu.make_async_copy(
            table_hbm.at[idx_v.at[v]], rows[b], gsem[b]).wait()

    def fire_write(v, b):
        pltpu.make_async_copy(
            rows[b].at[:, pl.ds(0, EMB)], out_hbm.at[base + v],
            wsem[b]).start()

    def wait_write(v, b):
        pltpu.make_async_copy(
            rows[b].at[:, pl.ds(0, EMB)], out_hbm.at[base + v],
            wsem[b]).wait()

    for b in range(NBUF):
        fire_gather(b, b)

    def group_body(g, carry):
        for b in range(NBUF):
            v = g * NBUF + b
            wait_gather(v, b)
            fire_write(v, b)
            wait_write(v, b)
            fire_gather(v + NBUF, b)
        return carry

    lax.fori_loop(0, NG - 1, group_body, 0)

    for b in range(NBUF):
        v = (NG - 1) * NBUF + b
        wait_gather(v, b)
        fire_write(v, b)
    for b in range(NBUF):
        v = (NG - 1) * NBUF + b
        wait_write(v, b)


def kernel(vocab_ids, node_embs):
    table = jnp.pad(node_embs, ((0, 7), (0, EMB)))
    return _gather_kernel(vocab_ids.astype(jnp.int32), table)
